# baseline (speedup 1.0000x reference)
import jax
import jax.numpy as jnp
from jax import lax
from jax.experimental import pallas as pl
from jax.experimental.pallas import tpu as pltpu
from jax.experimental.pallas import tpu_sc as plsc

N = 10000
ACC_ROWS = 10240
NC, NS = 2, 16
NW = NC * NS
CHUNK = 128
NCH = 82
EPW = NCH * CHUNK
NE_PAD = NW * EPW
ZBLK = 128
RPW = ACC_ROWS // NS

_MESH = plsc.VectorSubcoreMesh(core_axis_name="c", subcore_axis_name="s")
_SC_PARAMS = pltpu.CompilerParams(needs_layout_passes=False,
                                  use_tc_tiling_on_sc=False)


def _deg_body(dst_hbm, out_hbm, idx_v, hist_v):
    c = lax.axis_index("c")
    s = lax.axis_index("s")
    wid = s * NC + c
    pltpu.sync_copy(dst_hbm.at[wid], idx_v)
    zeros16 = jnp.zeros((16,), jnp.float32)

    def zbody(i, carry):
        hist_v[pl.ds(i * 16, 16)] = zeros16
        return carry

    lax.fori_loop(0, ACC_ROWS // 16, zbody, 0)
    ones16 = jnp.ones((16,), jnp.float32)

    def ebody(j, carry):
        idx = idx_v[pl.ds(j * 16, 16)]
        plsc.addupdate_scatter(hist_v, [idx], ones16)
        return carry

    lax.fori_loop(0, EPW // 16, ebody, 0)
    pltpu.sync_copy(hist_v, out_hbm.at[wid])


_deg_kernel = pl.kernel(
    _deg_body,
    out_type=jax.ShapeDtypeStruct((NW, ACC_ROWS), jnp.float32),
    mesh=_MESH,
    compiler_params=_SC_PARAMS,
    scratch_types=[
        pltpu.VMEM((EPW,), jnp.int32),
        pltpu.VMEM((ACC_ROWS,), jnp.float32),
    ],
)


def _make_agg(D):
    def body(table_hbm, src_hbm, dst_hbm, zrow_hbm, out_hbm,
             src_v, dst_v, rows0, acc_sh, sem0):
        c = lax.axis_index("c")
        s = lax.axis_index("s")
        wid = s * NC + c
        pltpu.sync_copy(src_hbm.at[wid], src_v)
        pltpu.sync_copy(dst_hbm.at[wid], dst_v)

        pltpu.sync_copy(zrow_hbm, rows0.at[pl.ds(0, ZBLK)])
        r0 = s * RPW
        for k in range(RPW // ZBLK):
            pltpu.sync_copy(rows0.at[pl.ds(0, ZBLK)],
                            acc_sh.at[pl.ds(r0 + k * ZBLK, ZBLK)])
        plsc.subcore_barrier()

        def ebody(j, carry):
            pltpu.async_copy(table_hbm.at[src_v.at[j]], rows0, sem0).wait()
            pltpu.sync_copy(rows0, acc_sh.at[dst_v.at[j]], add=True)
            return carry

        lax.fori_loop(0, NCH, ebody, 0)
        plsc.subcore_barrier()

        for k in range(RPW // ZBLK):
            pltpu.sync_copy(acc_sh.at[pl.ds(r0 + k * ZBLK, ZBLK)],
                            rows0.at[pl.ds(0, ZBLK)])
            pltpu.sync_copy(rows0.at[pl.ds(0, ZBLK)],
                            out_hbm.at[c].at[pl.ds(r0 + k * ZBLK, ZBLK)])

    return pl.kernel(
        body,
        out_type=jax.ShapeDtypeStruct((NC, ACC_ROWS, D), jnp.float32),
        mesh=_MESH,
        compiler_params=_SC_PARAMS,
        scratch_types=[
            pltpu.VMEM((NCH, CHUNK), jnp.int32),
            pltpu.VMEM((NCH, CHUNK), jnp.int32),
            pltpu.VMEM((CHUNK, D), jnp.float32),
            pltpu.VMEM_SHARED((ACC_ROWS, D), jnp.float32),
            pltpu.SemaphoreType.DMA,
        ],
    )


_agg64 = _make_agg(64)
_agg32 = _make_agg(32)


def _tc1_body(parts_ref, x_ref, w1_ref, dinv_ref, h1s_ref):
    deg = jnp.sum(parts_ref[...], axis=0)
    dinv = lax.rsqrt(jnp.maximum(deg, 1.0))
    dinv_ref[...] = dinv
    h1 = jnp.dot(x_ref[...], w1_ref[...], preferred_element_type=jnp.float32)
    h1s_ref[...] = h1 * dinv[:N][:, None]


_tc1 = pl.pallas_call(
    _tc1_body,
    out_shape=(
        jax.ShapeDtypeStruct((ACC_ROWS,), jnp.float32),
        jax.ShapeDtypeStruct((N, 64), jnp.float32),
    ),
)


def _tc2_body(p_ref, dinv_ref, b1_ref, w2_ref, h2s_ref):
    agg = (p_ref[0] + p_ref[1])[:N]
    dinv = dinv_ref[...][:N][:, None]
    hidden1 = agg * dinv + b1_ref[...][None, :]
    h2 = jnp.dot(hidden1, w2_ref[...], preferred_element_type=jnp.float32)
    h2s_ref[...] = h2 * dinv


_tc2 = pl.pallas_call(
    _tc2_body,
    out_shape=jax.ShapeDtypeStruct((N, 32), jnp.float32),
)


def _tc3_body(p_ref, dinv_ref, b2_ref, z_ref):
    agg = (p_ref[0] + p_ref[1])[:N]
    z_ref[...] = agg * dinv_ref[...][:N][:, None] + b2_ref[...][None, :]


_tc3 = pl.pallas_call(
    _tc3_body,
    out_shape=jax.ShapeDtypeStruct((N, 32), jnp.float32),
)


@jax.jit
def kernel(x, adj, W1, b1, W2, b2, W3, b3):
    n = x.shape[0]
    loop = jnp.arange(n, dtype=jnp.int32)
    src = jnp.concatenate([adj[0].astype(jnp.int32), loop])
    dst = jnp.concatenate([adj[1].astype(jnp.int32), loop])
    pad = NE_PAD - src.shape[0]
    src = jnp.concatenate([src, jnp.zeros((pad,), jnp.int32)])
    dst = jnp.concatenate([dst, jnp.full((pad,), N, jnp.int32)])
    src3 = src.reshape(NW, NCH, CHUNK)
    dst3 = dst.reshape(NW, NCH, CHUNK)
    dst2 = dst.reshape(NW, EPW)
    z64 = jnp.zeros((ZBLK, 64), jnp.float32)
    z32 = jnp.zeros((ZBLK, 32), jnp.float32)

    parts = _deg_kernel(dst2)
    dinv, h1s = _tc1(parts, x, W1)
    p1 = _agg64(h1s, src3, dst3, z64)
    h2s = _tc2(p1, dinv, b1, W2)
    p2 = _agg32(h2s, src3, dst3, z32)
    z = _tc3(p2, dinv, b2)
    return z


# speedup vs baseline: 1.0005x; 1.0005x over previous
import jax
import jax.numpy as jnp
from jax import lax
from jax.experimental import pallas as pl
from jax.experimental.pallas import tpu as pltpu
from jax.experimental.pallas import tpu_sc as plsc

N = 10000
ACC_ROWS = 10240
NC, NS = 2, 16
NW = NC * NS
CHUNK = 128
NCH = 82
EPW = NCH * CHUNK
NE_PAD = NW * EPW
ZBLK = 128
RPW = ACC_ROWS // NS

_MESH = plsc.VectorSubcoreMesh(core_axis_name="c", subcore_axis_name="s")
_SC_PARAMS = pltpu.CompilerParams(needs_layout_passes=False,
                                  use_tc_tiling_on_sc=False)


def _deg_body(dst_hbm, out_hbm, idx_v, hist_v):
    c = lax.axis_index("c")
    s = lax.axis_index("s")
    wid = s * NC + c
    pltpu.sync_copy(dst_hbm.at[wid], idx_v)
    zeros16 = jnp.zeros((16,), jnp.float32)

    def zbody(i, carry):
        hist_v[pl.ds(i * 16, 16)] = zeros16
        return carry

    lax.fori_loop(0, ACC_ROWS // 16, zbody, 0)
    ones16 = jnp.ones((16,), jnp.float32)

    def ebody(j, carry):
        idx = idx_v[pl.ds(j * 16, 16)]
        plsc.addupdate_scatter(hist_v, [idx], ones16)
        return carry

    lax.fori_loop(0, EPW // 16, ebody, 0)
    pltpu.sync_copy(hist_v, out_hbm.at[wid])


_deg_kernel = pl.kernel(
    _deg_body,
    out_type=jax.ShapeDtypeStruct((NW, ACC_ROWS), jnp.float32),
    mesh=_MESH,
    compiler_params=_SC_PARAMS,
    scratch_types=[
        pltpu.VMEM((EPW,), jnp.int32),
        pltpu.VMEM((ACC_ROWS,), jnp.float32),
    ],
)


def _make_agg(D):
    def body(table_hbm, src_hbm, dst_hbm, zrow_hbm, out_hbm,
             src_v, dst_v, rows0, acc_sh, sem0):
        c = lax.axis_index("c")
        s = lax.axis_index("s")
        wid = s * NC + c
        pltpu.sync_copy(src_hbm.at[wid], src_v)
        pltpu.sync_copy(dst_hbm.at[wid], dst_v)

        pltpu.sync_copy(zrow_hbm, rows0)
        r0 = s * RPW
        for k in range(RPW // ZBLK):
            pltpu.sync_copy(rows0, acc_sh.at[pl.ds(r0 + k * ZBLK, ZBLK)])
        plsc.subcore_barrier()

        def ebody(j, carry):
            pltpu.async_copy(table_hbm.at[src_v.at[j]], rows0, sem0).wait()
            pltpu.sync_copy(rows0, acc_sh.at[dst_v.at[j]], add=True)
            return carry

        lax.fori_loop(0, NCH, ebody, 0)
        plsc.subcore_barrier()

        for k in range(RPW // ZBLK):
            pltpu.sync_copy(acc_sh.at[pl.ds(r0 + k * ZBLK, ZBLK)], rows0)
            pltpu.sync_copy(rows0, out_hbm.at[c].at[pl.ds(r0 + k * ZBLK, ZBLK)])

    return pl.kernel(
        body,
        out_type=jax.ShapeDtypeStruct((NC, ACC_ROWS, D), jnp.float32),
        mesh=_MESH,
        compiler_params=_SC_PARAMS,
        scratch_types=[
            pltpu.VMEM((NCH, CHUNK), jnp.int32),
            pltpu.VMEM((NCH, CHUNK), jnp.int32),
            pltpu.VMEM((CHUNK, D), jnp.float32),
            pltpu.VMEM_SHARED((ACC_ROWS, D), jnp.float32),
            pltpu.SemaphoreType.DMA,
        ],
    )


_agg64 = _make_agg(64)
_agg32 = _make_agg(32)


def _tc1_body(parts_ref, x_ref, w1_ref, dinv_ref, h1s_ref):
    deg = jnp.sum(parts_ref[...], axis=0)
    dinv = lax.rsqrt(jnp.maximum(deg, 1.0))
    dinv_ref[...] = dinv
    h1 = jnp.dot(x_ref[...], w1_ref[...], preferred_element_type=jnp.float32)
    h1s_ref[...] = h1 * dinv[:N][:, None]


_tc1 = pl.pallas_call(
    _tc1_body,
    out_shape=(
        jax.ShapeDtypeStruct((ACC_ROWS,), jnp.float32),
        jax.ShapeDtypeStruct((N, 64), jnp.float32),
    ),
)


def _tc2_body(p_ref, dinv_ref, b1_ref, w2_ref, h2s_ref):
    agg = (p_ref[0] + p_ref[1])[:N]
    dinv = dinv_ref[...][:N][:, None]
    hidden1 = agg * dinv + b1_ref[...][None, :]
    h2 = jnp.dot(hidden1, w2_ref[...], preferred_element_type=jnp.float32)
    h2s_ref[...] = h2 * dinv


_tc2 = pl.pallas_call(
    _tc2_body,
    out_shape=jax.ShapeDtypeStruct((N, 32), jnp.float32),
)


def _tc3_body(p_ref, dinv_ref, b2_ref, z_ref):
    agg = (p_ref[0] + p_ref[1])[:N]
    z_ref[...] = agg * dinv_ref[...][:N][:, None] + b2_ref[...][None, :]


_tc3 = pl.pallas_call(
    _tc3_body,
    out_shape=jax.ShapeDtypeStruct((N, 32), jnp.float32),
)


@jax.jit
def kernel(x, adj, W1, b1, W2, b2, W3, b3):
    n = x.shape[0]
    loop = jnp.arange(n, dtype=jnp.int32)
    src = jnp.concatenate([adj[0].astype(jnp.int32), loop])
    dst = jnp.concatenate([adj[1].astype(jnp.int32), loop])
    pad = NE_PAD - src.shape[0]
    src = jnp.concatenate([src, jnp.zeros((pad,), jnp.int32)])
    dst = jnp.concatenate([dst, jnp.full((pad,), N, jnp.int32)])
    src3 = src.reshape(NW, NCH, CHUNK)
    dst3 = dst.reshape(NW, NCH, CHUNK)
    dst2 = dst.reshape(NW, EPW)
    z64 = jnp.zeros((ZBLK, 64), jnp.float32)
    z32 = jnp.zeros((ZBLK, 32), jnp.float32)

    parts = _deg_kernel(dst2)
    dinv, h1s = _tc1(parts, x, W1)
    p1 = _agg64(h1s, src3, dst3, z64)
    h2s = _tc2(p1, dinv, b1, W2)
    p2 = _agg32(h2s, src3, dst3, z32)
    z = _tc3(p2, dinv, b2)
    return z


# speedup vs baseline: 1.3595x; 1.3589x over previous
import jax
import jax.numpy as jnp
from jax import lax
from jax.experimental import pallas as pl
from jax.experimental.pallas import tpu as pltpu
from jax.experimental.pallas import tpu_sc as plsc

N = 10000
ACC_ROWS = 10240
NC, NS = 2, 16
NW = NC * NS
CHUNK = 128
NCH = 81
EPW = NCH * CHUNK
NE_PAD = NW * EPW
ZBLK = 128
RPW = ACC_ROWS // NS

_MESH = plsc.VectorSubcoreMesh(core_axis_name="c", subcore_axis_name="s")
_SC_PARAMS = pltpu.CompilerParams(needs_layout_passes=False,
                                  use_tc_tiling_on_sc=False)


def _deg_body(dst_hbm, out_hbm, idx_v, hist_v):
    c = lax.axis_index("c")
    s = lax.axis_index("s")
    wid = s * NC + c
    pltpu.sync_copy(dst_hbm.at[wid], idx_v)
    zeros16 = jnp.zeros((16,), jnp.float32)

    def zbody(i, carry):
        hist_v[pl.ds(i * 16, 16)] = zeros16
        return carry

    lax.fori_loop(0, ACC_ROWS // 16, zbody, 0)
    ones16 = jnp.ones((16,), jnp.float32)

    def ebody(j, carry):
        idx = idx_v[pl.ds(j * 16, 16)]
        plsc.addupdate_scatter(hist_v, [idx], ones16)
        return carry

    lax.fori_loop(0, EPW // 16, ebody, 0)
    pltpu.sync_copy(hist_v, out_hbm.at[wid])


_deg_kernel = pl.kernel(
    _deg_body,
    out_type=jax.ShapeDtypeStruct((NW, ACC_ROWS), jnp.float32),
    mesh=_MESH,
    compiler_params=_SC_PARAMS,
    scratch_types=[
        pltpu.VMEM((EPW,), jnp.int32),
        pltpu.VMEM((ACC_ROWS,), jnp.float32),
    ],
)


def _make_agg(D):
    def body(table_hbm, src_hbm, dst_hbm, zrow_hbm, out_hbm,
             src_v, dst_v, rows0, acc_sh, sem0):
        c = lax.axis_index("c")
        s = lax.axis_index("s")
        wid = s * NC + c
        pltpu.sync_copy(src_hbm.at[wid], src_v)
        pltpu.sync_copy(dst_hbm.at[wid], dst_v)

        pltpu.sync_copy(zrow_hbm, rows0)
        r0 = s * RPW
        for k in range(RPW // ZBLK):
            pltpu.sync_copy(rows0, acc_sh.at[pl.ds(r0 + k * ZBLK, ZBLK)])
        plsc.subcore_barrier()

        def ebody(j, carry):
            pltpu.async_copy(table_hbm.at[src_v.at[j]], rows0, sem0).wait()
            pltpu.sync_copy(rows0, acc_sh.at[dst_v.at[j]], add=True)
            return carry

        lax.fori_loop(0, NCH, ebody, 0)
        plsc.subcore_barrier()

        for k in range(RPW // ZBLK):
            pltpu.sync_copy(acc_sh.at[pl.ds(r0 + k * ZBLK, ZBLK)], rows0)
            pltpu.sync_copy(rows0, out_hbm.at[c].at[pl.ds(r0 + k * ZBLK, ZBLK)])

    return pl.kernel(
        body,
        out_type=jax.ShapeDtypeStruct((NC, ACC_ROWS, D), jnp.float32),
        mesh=_MESH,
        compiler_params=_SC_PARAMS,
        scratch_types=[
            pltpu.VMEM((NCH, CHUNK), jnp.int32),
            pltpu.VMEM((NCH, CHUNK), jnp.int32),
            pltpu.VMEM((CHUNK, D), jnp.float32),
            pltpu.VMEM_SHARED((ACC_ROWS, D), jnp.float32),
            pltpu.SemaphoreType.DMA,
        ],
    )


_agg64 = _make_agg(64)
_agg32 = _make_agg(32)


def _tc1_body(parts_ref, x_ref, w1_ref, dinv_ref, h1s_ref):
    deg = jnp.sum(parts_ref[...], axis=0)
    dinv = lax.rsqrt(jnp.maximum(deg, 1.0))
    dinv_ref[...] = dinv
    h1 = jnp.dot(x_ref[...], w1_ref[...], preferred_element_type=jnp.float32)
    h1s_ref[...] = h1 * dinv[:N][:, None]


_tc1 = pl.pallas_call(
    _tc1_body,
    out_shape=(
        jax.ShapeDtypeStruct((ACC_ROWS,), jnp.float32),
        jax.ShapeDtypeStruct((N, 64), jnp.float32),
    ),
)


def _tc2_body(p_ref, dinv_ref, b1_ref, w2_ref, h2s_ref):
    agg = (p_ref[0] + p_ref[1])[:N]
    dinv = dinv_ref[...][:N][:, None]
    hidden1 = agg * dinv + b1_ref[...][None, :]
    h2 = jnp.dot(hidden1, w2_ref[...], preferred_element_type=jnp.float32)
    h2s_ref[...] = h2 * dinv


_tc2 = pl.pallas_call(
    _tc2_body,
    out_shape=jax.ShapeDtypeStruct((N, 32), jnp.float32),
)


def _tc3_body(p_ref, dinv_ref, b2_ref, z_ref):
    agg = (p_ref[0] + p_ref[1])[:N]
    z_ref[...] = agg * dinv_ref[...][:N][:, None] + b2_ref[...][None, :]


_tc3 = pl.pallas_call(
    _tc3_body,
    out_shape=jax.ShapeDtypeStruct((N, 32), jnp.float32),
)


@jax.jit
def kernel(x, adj, W1, b1, W2, b2, W3, b3):
    n = x.shape[0]
    loop = jnp.arange(n, dtype=jnp.int32)
    src = jnp.concatenate([adj[0].astype(jnp.int32), loop])
    dst = jnp.concatenate([adj[1].astype(jnp.int32), loop])
    pad = NE_PAD - src.shape[0]
    src = jnp.concatenate([src, jnp.zeros((pad,), jnp.int32)])
    dst = jnp.concatenate([dst, jnp.full((pad,), N, jnp.int32)])
    src3 = src.reshape(NW, NCH, CHUNK)
    dst3 = dst.reshape(NW, NCH, CHUNK)
    dst2 = dst.reshape(NW, EPW)
    z64 = jnp.zeros((ZBLK, 64), jnp.float32)
    z32 = jnp.zeros((ZBLK, 32), jnp.float32)

    parts = _deg_kernel(dst2)
    dinv, h1s = _tc1(parts, x, W1)
    p1 = _agg64(h1s, src3, dst3, z64)
    h2s = _tc2(p1, dinv, b1, W2)
    p2 = _agg32(h2s, src3, dst3, z32)
    z = _tc3(p2, dinv, b2)
    return z


# speedup vs baseline: 1.7760x; 1.3063x over previous
import jax
import jax.numpy as jnp
from jax import lax
from jax.experimental import pallas as pl
from jax.experimental.pallas import tpu as pltpu
from jax.experimental.pallas import tpu_sc as plsc

N = 10000
ACC_ROWS = 10240
NC, NS = 2, 16
NW = NC * NS
CHUNK = 128
NCH = 81
EPW = NCH * CHUNK
NE_PAD = NW * EPW
ZBLK = 128
RPW = ACC_ROWS // NS

_MESH = plsc.VectorSubcoreMesh(core_axis_name="c", subcore_axis_name="s")
_SC_PARAMS = pltpu.CompilerParams(needs_layout_passes=False,
                                  use_tc_tiling_on_sc=False)


def _deg_body(dst_hbm, out_hbm, idx_v, hist_v):
    c = lax.axis_index("c")
    s = lax.axis_index("s")
    wid = s * NC + c
    pltpu.sync_copy(dst_hbm.at[wid], idx_v)
    zeros16 = jnp.zeros((16,), jnp.float32)

    def zbody(i, carry):
        hist_v[pl.ds(i * 16, 16)] = zeros16
        return carry

    lax.fori_loop(0, ACC_ROWS // 16, zbody, 0)
    ones16 = jnp.ones((16,), jnp.float32)

    def ebody(j, carry):
        idx = idx_v[pl.ds(j * 16, 16)]
        plsc.addupdate_scatter(hist_v, [idx], ones16)
        return carry

    lax.fori_loop(0, EPW // 16, ebody, 0)
    pltpu.sync_copy(hist_v, out_hbm.at[wid])


_deg_kernel = pl.kernel(
    _deg_body,
    out_type=jax.ShapeDtypeStruct((NW, ACC_ROWS), jnp.float32),
    mesh=_MESH,
    compiler_params=_SC_PARAMS,
    scratch_types=[
        pltpu.VMEM((EPW,), jnp.int32),
        pltpu.VMEM((ACC_ROWS,), jnp.float32),
    ],
)


def _make_agg(D):
    def body(table_hbm, src_hbm, dst_hbm, zrow_hbm, out_hbm,
             src_v, dst_v, rows0, rows1, acc_sh, sem0, sem1):
        c = lax.axis_index("c")
        s = lax.axis_index("s")
        wid = s * NC + c
        pltpu.sync_copy(src_hbm.at[wid], src_v)
        pltpu.sync_copy(dst_hbm.at[wid], dst_v)

        pltpu.sync_copy(zrow_hbm, rows0)
        r0 = s * RPW
        for k in range(RPW // ZBLK):
            pltpu.sync_copy(rows0, acc_sh.at[pl.ds(r0 + k * ZBLK, ZBLK)])
        plsc.subcore_barrier()

        pltpu.async_copy(table_hbm.at[src_v.at[0]], rows0, sem0)
        pltpu.async_copy(table_hbm.at[src_v.at[1]], rows1, sem1)

        def ebody(i, carry):
            g = 2 * i
            pltpu.make_async_copy(table_hbm.at[src_v.at[g]], rows0, sem0).wait()
            pltpu.sync_copy(rows0, acc_sh.at[dst_v.at[g]], add=True)
            pltpu.async_copy(table_hbm.at[src_v.at[g + 2]], rows0, sem0)
            pltpu.make_async_copy(table_hbm.at[src_v.at[g + 1]], rows1, sem1).wait()
            pltpu.sync_copy(rows1, acc_sh.at[dst_v.at[g + 1]], add=True)
            pltpu.async_copy(table_hbm.at[src_v.at[g + 3]], rows1, sem1)
            return carry

        lax.fori_loop(0, (NCH - 3) // 2, ebody, 0)
        pltpu.make_async_copy(table_hbm.at[src_v.at[NCH - 3]], rows0, sem0).wait()
        pltpu.sync_copy(rows0, acc_sh.at[dst_v.at[NCH - 3]], add=True)
        pltpu.make_async_copy(table_hbm.at[src_v.at[NCH - 2]], rows1, sem1).wait()
        pltpu.sync_copy(rows1, acc_sh.at[dst_v.at[NCH - 2]], add=True)
        pltpu.async_copy(table_hbm.at[src_v.at[NCH - 1]], rows0, sem0).wait()
        pltpu.sync_copy(rows0, acc_sh.at[dst_v.at[NCH - 1]], add=True)
        plsc.subcore_barrier()

        for k in range(RPW // ZBLK):
            pltpu.sync_copy(acc_sh.at[pl.ds(r0 + k * ZBLK, ZBLK)], rows0)
            pltpu.sync_copy(rows0, out_hbm.at[c].at[pl.ds(r0 + k * ZBLK, ZBLK)])

    return pl.kernel(
        body,
        out_type=jax.ShapeDtypeStruct((NC, ACC_ROWS, D), jnp.float32),
        mesh=_MESH,
        compiler_params=_SC_PARAMS,
        scratch_types=[
            pltpu.VMEM((NCH, CHUNK), jnp.int32),
            pltpu.VMEM((NCH, CHUNK), jnp.int32),
            pltpu.VMEM((CHUNK, D), jnp.float32),
            pltpu.VMEM((CHUNK, D), jnp.float32),
            pltpu.VMEM_SHARED((ACC_ROWS, D), jnp.float32),
            pltpu.SemaphoreType.DMA,
            pltpu.SemaphoreType.DMA,
        ],
    )


_agg64 = _make_agg(64)
_agg32 = _make_agg(32)


def _tc1_body(parts_ref, x_ref, w1_ref, dinv_ref, h1s_ref):
    deg = jnp.sum(parts_ref[...], axis=0)
    dinv = lax.rsqrt(jnp.maximum(deg, 1.0))
    dinv_ref[...] = dinv
    h1 = jnp.dot(x_ref[...], w1_ref[...], preferred_element_type=jnp.float32)
    h1s_ref[...] = h1 * dinv[:N][:, None]


_tc1 = pl.pallas_call(
    _tc1_body,
    out_shape=(
        jax.ShapeDtypeStruct((ACC_ROWS,), jnp.float32),
        jax.ShapeDtypeStruct((N, 64), jnp.float32),
    ),
)


def _tc2_body(p_ref, dinv_ref, b1_ref, w2_ref, h2s_ref):
    agg = (p_ref[0] + p_ref[1])[:N]
    dinv = dinv_ref[...][:N][:, None]
    hidden1 = agg * dinv + b1_ref[...][None, :]
    h2 = jnp.dot(hidden1, w2_ref[...], preferred_element_type=jnp.float32)
    h2s_ref[...] = h2 * dinv


_tc2 = pl.pallas_call(
    _tc2_body,
    out_shape=jax.ShapeDtypeStruct((N, 32), jnp.float32),
)


def _tc3_body(p_ref, dinv_ref, b2_ref, z_ref):
    agg = (p_ref[0] + p_ref[1])[:N]
    z_ref[...] = agg * dinv_ref[...][:N][:, None] + b2_ref[...][None, :]


_tc3 = pl.pallas_call(
    _tc3_body,
    out_shape=jax.ShapeDtypeStruct((N, 32), jnp.float32),
)


@jax.jit
def kernel(x, adj, W1, b1, W2, b2, W3, b3):
    n = x.shape[0]
    loop = jnp.arange(n, dtype=jnp.int32)
    src = jnp.concatenate([adj[0].astype(jnp.int32), loop])
    dst = jnp.concatenate([adj[1].astype(jnp.int32), loop])
    pad = NE_PAD - src.shape[0]
    src = jnp.concatenate([src, jnp.zeros((pad,), jnp.int32)])
    dst = jnp.concatenate([dst, jnp.full((pad,), N, jnp.int32)])
    src3 = src.reshape(NW, NCH, CHUNK)
    dst3 = dst.reshape(NW, NCH, CHUNK)
    dst2 = dst.reshape(NW, EPW)
    z64 = jnp.zeros((ZBLK, 64), jnp.float32)
    z32 = jnp.zeros((ZBLK, 32), jnp.float32)

    parts = _deg_kernel(dst2)
    dinv, h1s = _tc1(parts, x, W1)
    p1 = _agg64(h1s, src3, dst3, z64)
    h2s = _tc2(p1, dinv, b1, W2)
    p2 = _agg32(h2s, src3, dst3, z32)
    z = _tc3(p2, dinv, b2)
    return z


# speedup vs baseline: 1.9117x; 1.0764x over previous
import jax
import jax.numpy as jnp
from jax import lax
from jax.experimental import pallas as pl
from jax.experimental.pallas import tpu as pltpu
from jax.experimental.pallas import tpu_sc as plsc

N = 10000
ACC_ROWS = 10240
NC, NS = 2, 16
NW = NC * NS
CHUNK = 128
NCH = 81
EPW = NCH * CHUNK
NE_PAD = NW * EPW
ZBLK = 128
RPW = ACC_ROWS // NS

_MESH = plsc.VectorSubcoreMesh(core_axis_name="c", subcore_axis_name="s")
_SC_PARAMS = pltpu.CompilerParams(needs_layout_passes=False,
                                  use_tc_tiling_on_sc=False)


def _deg_body(dst_hbm, out_hbm, idx_v, hist_v):
    c = lax.axis_index("c")
    s = lax.axis_index("s")
    wid = s * NC + c
    pltpu.sync_copy(dst_hbm.at[wid], idx_v)
    zeros16 = jnp.zeros((16,), jnp.float32)

    def zbody(i, carry):
        hist_v[pl.ds(i * 16, 16)] = zeros16
        return carry

    lax.fori_loop(0, ACC_ROWS // 16, zbody, 0)
    ones16 = jnp.ones((16,), jnp.float32)

    def ebody(j, carry):
        idx = idx_v[pl.ds(j * 16, 16)]
        plsc.addupdate_scatter(hist_v, [idx], ones16)
        return carry

    lax.fori_loop(0, EPW // 16, ebody, 0)
    pltpu.sync_copy(hist_v, out_hbm.at[wid])


_deg_kernel = pl.kernel(
    _deg_body,
    out_type=jax.ShapeDtypeStruct((NW, ACC_ROWS), jnp.float32),
    mesh=_MESH,
    compiler_params=_SC_PARAMS,
    scratch_types=[
        pltpu.VMEM((EPW,), jnp.int32),
        pltpu.VMEM((ACC_ROWS,), jnp.float32),
    ],
)


def _make_agg(D):
    def body(table_hbm, src_hbm, dst_hbm, zrow_hbm, out_hbm,
             src_v, dst_v, rows0, rows1, rows2, acc_sh, sem0, sem1, sem2):
        c = lax.axis_index("c")
        s = lax.axis_index("s")
        wid = s * NC + c
        pltpu.sync_copy(src_hbm.at[wid], src_v)
        pltpu.sync_copy(dst_hbm.at[wid], dst_v)

        pltpu.sync_copy(zrow_hbm, rows0)
        r0 = s * RPW
        for k in range(RPW // ZBLK):
            pltpu.sync_copy(rows0, acc_sh.at[pl.ds(r0 + k * ZBLK, ZBLK)])
        plsc.subcore_barrier()

        pltpu.async_copy(table_hbm.at[src_v.at[0]], rows0, sem0)
        pltpu.async_copy(table_hbm.at[src_v.at[1]], rows1, sem1)
        pltpu.async_copy(table_hbm.at[src_v.at[2]], rows2, sem2)

        def ebody(i, carry):
            g = 3 * i
            pltpu.make_async_copy(table_hbm.at[src_v.at[g]], rows0, sem0).wait()
            pltpu.sync_copy(rows0, acc_sh.at[dst_v.at[g]], add=True)
            pltpu.async_copy(table_hbm.at[src_v.at[g + 3]], rows0, sem0)
            pltpu.make_async_copy(table_hbm.at[src_v.at[g + 1]], rows1, sem1).wait()
            pltpu.sync_copy(rows1, acc_sh.at[dst_v.at[g + 1]], add=True)
            pltpu.async_copy(table_hbm.at[src_v.at[g + 4]], rows1, sem1)
            pltpu.make_async_copy(table_hbm.at[src_v.at[g + 2]], rows2, sem2).wait()
            pltpu.sync_copy(rows2, acc_sh.at[dst_v.at[g + 2]], add=True)
            pltpu.async_copy(table_hbm.at[src_v.at[g + 5]], rows2, sem2)
            return carry

        lax.fori_loop(0, NCH // 3 - 1, ebody, 0)
        pltpu.make_async_copy(table_hbm.at[src_v.at[NCH - 3]], rows0, sem0).wait()
        pltpu.sync_copy(rows0, acc_sh.at[dst_v.at[NCH - 3]], add=True)
        pltpu.make_async_copy(table_hbm.at[src_v.at[NCH - 2]], rows1, sem1).wait()
        pltpu.sync_copy(rows1, acc_sh.at[dst_v.at[NCH - 2]], add=True)
        pltpu.make_async_copy(table_hbm.at[src_v.at[NCH - 1]], rows2, sem2).wait()
        pltpu.sync_copy(rows2, acc_sh.at[dst_v.at[NCH - 1]], add=True)
        plsc.subcore_barrier()

        for k in range(RPW // ZBLK):
            pltpu.sync_copy(acc_sh.at[pl.ds(r0 + k * ZBLK, ZBLK)], rows0)
            pltpu.sync_copy(rows0, out_hbm.at[c].at[pl.ds(r0 + k * ZBLK, ZBLK)])

    return pl.kernel(
        body,
        out_type=jax.ShapeDtypeStruct((NC, ACC_ROWS, D), jnp.float32),
        mesh=_MESH,
        compiler_params=_SC_PARAMS,
        scratch_types=[
            pltpu.VMEM((NCH, CHUNK), jnp.int32),
            pltpu.VMEM((NCH, CHUNK), jnp.int32),
            pltpu.VMEM((CHUNK, D), jnp.float32),
            pltpu.VMEM((CHUNK, D), jnp.float32),
            pltpu.VMEM((CHUNK, D), jnp.float32),
            pltpu.VMEM_SHARED((ACC_ROWS, D), jnp.float32),
            pltpu.SemaphoreType.DMA,
            pltpu.SemaphoreType.DMA,
            pltpu.SemaphoreType.DMA,
        ],
    )


_agg64 = _make_agg(64)
_agg32 = _make_agg(32)


def _tc1_body(parts_ref, x_ref, w1_ref, dinv_ref, h1s_ref):
    deg = jnp.sum(parts_ref[...], axis=0)
    dinv = lax.rsqrt(jnp.maximum(deg, 1.0))
    dinv_ref[...] = dinv
    h1 = jnp.dot(x_ref[...], w1_ref[...], preferred_element_type=jnp.float32)
    h1s_ref[...] = h1 * dinv[:N][:, None]


_tc1 = pl.pallas_call(
    _tc1_body,
    out_shape=(
        jax.ShapeDtypeStruct((ACC_ROWS,), jnp.float32),
        jax.ShapeDtypeStruct((N, 64), jnp.float32),
    ),
)


def _tc2_body(p_ref, dinv_ref, b1_ref, w2_ref, h2s_ref):
    agg = (p_ref[0] + p_ref[1])[:N]
    dinv = dinv_ref[...][:N][:, None]
    hidden1 = agg * dinv + b1_ref[...][None, :]
    h2 = jnp.dot(hidden1, w2_ref[...], preferred_element_type=jnp.float32)
    h2s_ref[...] = h2 * dinv


_tc2 = pl.pallas_call(
    _tc2_body,
    out_shape=jax.ShapeDtypeStruct((N, 32), jnp.float32),
)


def _tc3_body(p_ref, dinv_ref, b2_ref, z_ref):
    agg = (p_ref[0] + p_ref[1])[:N]
    z_ref[...] = agg * dinv_ref[...][:N][:, None] + b2_ref[...][None, :]


_tc3 = pl.pallas_call(
    _tc3_body,
    out_shape=jax.ShapeDtypeStruct((N, 32), jnp.float32),
)


@jax.jit
def kernel(x, adj, W1, b1, W2, b2, W3, b3):
    n = x.shape[0]
    loop = jnp.arange(n, dtype=jnp.int32)
    src = jnp.concatenate([adj[0].astype(jnp.int32), loop])
    dst = jnp.concatenate([adj[1].astype(jnp.int32), loop])
    pad = NE_PAD - src.shape[0]
    src = jnp.concatenate([src, jnp.zeros((pad,), jnp.int32)])
    dst = jnp.concatenate([dst, jnp.full((pad,), N, jnp.int32)])
    src3 = src.reshape(NW, NCH, CHUNK)
    dst3 = dst.reshape(NW, NCH, CHUNK)
    dst2 = dst.reshape(NW, EPW)
    z64 = jnp.zeros((ZBLK, 64), jnp.float32)
    z32 = jnp.zeros((ZBLK, 32), jnp.float32)

    parts = _deg_kernel(dst2)
    dinv, h1s = _tc1(parts, x, W1)
    p1 = _agg64(h1s, src3, dst3, z64)
    h2s = _tc2(p1, dinv, b1, W2)
    p2 = _agg32(h2s, src3, dst3, z32)
    z = _tc3(p2, dinv, b2)
    return z
